# Initial kernel scaffold; baseline (speedup 1.0000x reference)
import jax
import jax.numpy as jnp
from jax.experimental import pallas as pl


def kernel(x, y, mem_x, mem_y):
    raise NotImplementedError("write your pallas kernel here")


# speedup vs baseline: 13.6944x; 13.6944x over previous
import jax
import jax.numpy as jnp
from jax.experimental import pallas as pl
from jax.experimental.pallas import tpu as pltpu

_CAP = 200000
_D = 128
_B = 512
_NC = 100
_CHUNK = 1024
_NCH = 196
_PAD = _NCH * _CHUNK - _CAP
_PADV = 255
_BIG = 1 << 20


def _decide_body(y_ref, hb_ref, lb_ref, memy_ref, memy_out, ridx_out,
                 cc_ref, cnt_ref, ltri_ref, utri_ref):
    lane_ch = jax.lax.broadcasted_iota(jnp.int32, (1, _CHUNK), 1)
    lane_cnt = jax.lax.broadcasted_iota(jnp.int32, (1, 128), 1)
    lane_b = jax.lax.broadcasted_iota(jnp.int32, (1, _B), 1)
    sub_iota = jax.lax.broadcasted_iota(jnp.int32, (_NCH, 1), 0)
    hp = jax.lax.Precision.HIGHEST

    ltri_ref[...] = (
        jax.lax.broadcasted_iota(jnp.int32, (_NCH, _NCH), 1)
        <= jax.lax.broadcasted_iota(jnp.int32, (_NCH, _NCH), 0)
    ).astype(jnp.float32)
    utri_ref[...] = (
        jax.lax.broadcasted_iota(jnp.int32, (_CHUNK, _CHUNK), 0)
        <= jax.lax.broadcasted_iota(jnp.int32, (_CHUNK, _CHUNK), 1)
    ).astype(jnp.float32)

    memy_out[...] = memy_ref[...]
    ridx_out[...] = jnp.zeros((1, _B), jnp.int32)
    cc_ref[...] = jnp.zeros((_NCH, 128), jnp.int32)
    cnt_ref[...] = jnp.full((1, 128), -1, jnp.int32)

    def init_class(c, _):
        match = (memy_ref[...] == c).astype(jnp.int32)
        col = jnp.sum(match, axis=1, keepdims=True)
        cc_ref[...] = jnp.where(lane_cnt == c, col, cc_ref[...])
        total = jnp.sum(col)
        cnt_ref[...] = jnp.where(lane_cnt == c, total, cnt_ref[...])
        return 0

    jax.lax.fori_loop(0, _NC, init_class, 0)

    def body(i, _):
        cnt = cnt_ref[...]
        ncand = jnp.max(cnt)
        c = jnp.min(jnp.where(cnt == ncand, lane_cnt, _BIG))

        mu = jnp.maximum(ncand, 1).astype(jnp.uint32)
        hbi = hb_ref[i].astype(jnp.uint32)
        lbi = lb_ref[i].astype(jnp.uint32)
        mult = jax.lax.rem(jnp.uint32(65536), mu)
        mult = jax.lax.rem(mult * mult, mu)
        roff = jax.lax.rem(hbi, mu) * mult + jax.lax.rem(lbi, mu)
        r = jax.lax.rem(roff, mu).astype(jnp.int32)

        cc = cc_ref[...]
        col = jnp.sum(jnp.where(lane_cnt == c, cc, 0), axis=1,
                      keepdims=True).astype(jnp.float32)
        cs = jnp.dot(ltri_ref[...], col, precision=hp)
        rf = r.astype(jnp.float32)
        k = jnp.min(jnp.where(cs > rf, sub_iota, _BIG))
        prior = jnp.max(jnp.where(cs <= rf, cs, 0.0))
        rloc = r - prior.astype(jnp.int32)

        vals = memy_out[pl.ds(k, 1), :]
        matchk = (vals == c).astype(jnp.float32)
        lcs = jnp.dot(matchk, utri_ref[...], precision=hp)
        tgt = (rloc + 1).astype(jnp.float32)
        pos = jnp.min(jnp.where(lcs == tgt, lane_ch, _BIG))
        ridx = k * _CHUNK + pos

        yi = y_ref[i]
        memy_out[pl.ds(k, 1), :] = jnp.where(lane_ch == pos, yi, vals)

        dlt = ((lane_cnt == yi).astype(jnp.int32)
               - (lane_cnt == c).astype(jnp.int32))
        cnt_ref[...] = cnt + dlt
        cc_ref[...] = cc + jnp.where(sub_iota == k, dlt, 0)

        ridx_out[...] = jnp.where(lane_b == i, ridx, ridx_out[...])
        return 0

    jax.lax.fori_loop(0, _B, body, 0)


def _scatter_body(ridx_ref, x_ref, memx_ref, out_ref):
    del ridx_ref, memx_ref
    out_ref[...] = x_ref[...]


def kernel(x, y, mem_x, mem_y):
    ydt = mem_y.dtype
    y_i = y.astype(jnp.int32)
    memy_i = mem_y.astype(jnp.int32)

    base = jax.random.key(1)
    keys = jax.vmap(lambda i: jax.random.fold_in(base, i))(jnp.arange(_B))
    splits = jax.vmap(jax.random.split)(keys)
    hb = jax.vmap(lambda k: jax.random.bits(k, (), jnp.uint32))(splits[:, 0])
    lb = jax.vmap(lambda k: jax.random.bits(k, (), jnp.uint32))(splits[:, 1])
    hb_i = hb.astype(jnp.int32)
    lb_i = lb.astype(jnp.int32)

    memy_p = jnp.concatenate(
        [memy_i, jnp.full((_PAD,), _PADV, jnp.int32)]).reshape(_NCH, _CHUNK)

    memy_new, ridx2d = pl.pallas_call(
        _decide_body,
        out_shape=(jax.ShapeDtypeStruct((_NCH, _CHUNK), jnp.int32),
                   jax.ShapeDtypeStruct((1, _B), jnp.int32)),
        in_specs=[pl.BlockSpec(memory_space=pltpu.SMEM),
                  pl.BlockSpec(memory_space=pltpu.SMEM),
                  pl.BlockSpec(memory_space=pltpu.SMEM),
                  pl.BlockSpec(memory_space=pltpu.VMEM)],
        out_specs=(pl.BlockSpec(memory_space=pltpu.VMEM),
                   pl.BlockSpec(memory_space=pltpu.VMEM)),
        scratch_shapes=[pltpu.VMEM((_NCH, 128), jnp.int32),
                        pltpu.VMEM((1, 128), jnp.int32),
                        pltpu.VMEM((_NCH, _NCH), jnp.float32),
                        pltpu.VMEM((_CHUNK, _CHUNK), jnp.float32)],
    )(y_i, hb_i, lb_i, memy_p)

    ridx = ridx2d.reshape(_B)

    x3 = x.reshape(_B, 1, _D)
    memx3 = mem_x.reshape(_CAP, 1, _D)
    out = pl.pallas_call(
        _scatter_body,
        grid_spec=pltpu.PrefetchScalarGridSpec(
            num_scalar_prefetch=1,
            grid=(_B,),
            in_specs=[pl.BlockSpec((1, 1, _D), lambda i, ridx_s: (i, 0, 0)),
                      pl.BlockSpec(memory_space=pl.ANY)],
            out_specs=pl.BlockSpec((1, 1, _D),
                                   lambda i, ridx_s: (ridx_s[i], 0, 0)),
        ),
        out_shape=jax.ShapeDtypeStruct((_CAP, 1, _D), x.dtype),
        input_output_aliases={2: 0},
    )(ridx, x3, memx3)

    mem_x_out = out.reshape(_CAP, _D)
    mem_y_out = memy_new.reshape(-1)[:_CAP].astype(ydt)
    return mem_x_out, mem_y_out


# speedup vs baseline: 23.8335x; 1.7404x over previous
import jax
import jax.numpy as jnp
from jax.experimental import pallas as pl
from jax.experimental.pallas import tpu as pltpu

_CAP = 200000
_D = 128
_B = 512
_NC = 100
_CHUNK = 1024
_NCH = 196
_PAD = _NCH * _CHUNK - _CAP
_PADV = 255
_BIG = 1 << 20


def _decide_body(y_ref, hb_ref, lb_ref, memy_ref, memy_out, ridx_out,
                 cc_ref, cnt_ref, ltri_ref, utri_ref, l8_ref):
    lane_ch = jax.lax.broadcasted_iota(jnp.int32, (1, _CHUNK), 1)
    lane_cnt = jax.lax.broadcasted_iota(jnp.int32, (1, 128), 1)
    lane_b = jax.lax.broadcasted_iota(jnp.int32, (1, _B), 1)
    sub_iota = jax.lax.broadcasted_iota(jnp.int32, (_NCH, 1), 0)
    hp = jax.lax.Precision.HIGHEST

    ltri_ref[...] = (
        jax.lax.broadcasted_iota(jnp.int32, (_NCH, _NCH), 1)
        <= jax.lax.broadcasted_iota(jnp.int32, (_NCH, _NCH), 0)
    ).astype(jnp.float32)
    utri_ref[...] = (
        jax.lax.broadcasted_iota(jnp.int32, (128, 128), 0)
        <= jax.lax.broadcasted_iota(jnp.int32, (128, 128), 1)
    ).astype(jnp.float32)
    l8_ref[...] = (
        jax.lax.broadcasted_iota(jnp.int32, (8, 8), 1)
        <= jax.lax.broadcasted_iota(jnp.int32, (8, 8), 0)
    ).astype(jnp.float32)
    sub8 = jax.lax.broadcasted_iota(jnp.int32, (8, 1), 0)
    flat8 = (jax.lax.broadcasted_iota(jnp.int32, (8, 128), 0) * 128
             + jax.lax.broadcasted_iota(jnp.int32, (8, 128), 1))

    memy_out[...] = memy_ref[...]
    ridx_out[...] = jnp.zeros((1, _B), jnp.int32)
    cc_ref[...] = jnp.zeros((_NCH, 128), jnp.int32)
    cnt_ref[...] = jnp.full((1, 128), -1, jnp.int32)

    def init_class(c, _):
        match = (memy_ref[...] == c).astype(jnp.int32)
        col = jnp.sum(match, axis=1, keepdims=True)
        cc_ref[...] = jnp.where(lane_cnt == c, col, cc_ref[...])
        total = jnp.sum(col)
        cnt_ref[...] = jnp.where(lane_cnt == c, total, cnt_ref[...])
        return 0

    jax.lax.fori_loop(0, _NC, init_class, 0)

    def body(i, _):
        cnt = cnt_ref[...]
        ncand = jnp.max(cnt)
        c = jnp.min(jnp.where(cnt == ncand, lane_cnt, _BIG))

        mu = jnp.maximum(ncand, 1).astype(jnp.uint32)
        hbi = hb_ref[i].astype(jnp.uint32)
        lbi = lb_ref[i].astype(jnp.uint32)
        mult = jax.lax.rem(jnp.uint32(65536), mu)
        mult = jax.lax.rem(mult * mult, mu)
        roff = jax.lax.rem(hbi, mu) * mult + jax.lax.rem(lbi, mu)
        r = jax.lax.rem(roff, mu).astype(jnp.int32)

        cc = cc_ref[...]
        col = jnp.sum(jnp.where(lane_cnt == c, cc, 0), axis=1,
                      keepdims=True).astype(jnp.float32)
        cs = jnp.dot(ltri_ref[...], col, precision=hp)
        rf = r.astype(jnp.float32)
        k = jnp.min(jnp.where(cs > rf, sub_iota, _BIG))
        prior = jnp.max(jnp.where(cs <= rf, cs, 0.0))
        rloc = r - prior.astype(jnp.int32)

        vals = memy_out[pl.ds(k, 1), :].reshape(8, 128)
        m8 = (vals == c).astype(jnp.float32)
        rowtot = jnp.sum(m8, axis=1, keepdims=True)
        cs8 = jnp.dot(l8_ref[...], rowtot)
        lcs = jnp.dot(m8, utri_ref[...])
        full = lcs + (cs8 - rowtot)
        tgt = (rloc + 1).astype(jnp.float32)
        pos = jnp.min(jnp.where(full == tgt, flat8, _BIG))
        ridx = k * _CHUNK + pos

        yi = y_ref[i]
        memy_out[pl.ds(k, 1), :] = jnp.where(
            flat8 == pos, yi, vals).reshape(1, _CHUNK)

        dlt = ((lane_cnt == yi).astype(jnp.int32)
               - (lane_cnt == c).astype(jnp.int32))
        cnt_ref[...] = cnt + dlt
        cc_ref[...] = cc + jnp.where(sub_iota == k, dlt, 0)

        ridx_out[...] = jnp.where(lane_b == i, ridx, ridx_out[...])
        return 0

    jax.lax.fori_loop(0, _B, body, 0)


def _scatter_body(ridx_ref, x_ref, memx_ref, out_ref):
    del ridx_ref, memx_ref
    out_ref[...] = x_ref[...]


def kernel(x, y, mem_x, mem_y):
    ydt = mem_y.dtype
    y_i = y.astype(jnp.int32)
    memy_i = mem_y.astype(jnp.int32)

    base = jax.random.key(1)
    keys = jax.vmap(lambda i: jax.random.fold_in(base, i))(jnp.arange(_B))
    splits = jax.vmap(jax.random.split)(keys)
    hb = jax.vmap(lambda k: jax.random.bits(k, (), jnp.uint32))(splits[:, 0])
    lb = jax.vmap(lambda k: jax.random.bits(k, (), jnp.uint32))(splits[:, 1])
    hb_i = hb.astype(jnp.int32)
    lb_i = lb.astype(jnp.int32)

    memy_p = jnp.concatenate(
        [memy_i, jnp.full((_PAD,), _PADV, jnp.int32)]).reshape(_NCH, _CHUNK)

    memy_new, ridx2d = pl.pallas_call(
        _decide_body,
        out_shape=(jax.ShapeDtypeStruct((_NCH, _CHUNK), jnp.int32),
                   jax.ShapeDtypeStruct((1, _B), jnp.int32)),
        in_specs=[pl.BlockSpec(memory_space=pltpu.SMEM),
                  pl.BlockSpec(memory_space=pltpu.SMEM),
                  pl.BlockSpec(memory_space=pltpu.SMEM),
                  pl.BlockSpec(memory_space=pltpu.VMEM)],
        out_specs=(pl.BlockSpec(memory_space=pltpu.VMEM),
                   pl.BlockSpec(memory_space=pltpu.VMEM)),
        scratch_shapes=[pltpu.VMEM((_NCH, 128), jnp.int32),
                        pltpu.VMEM((1, 128), jnp.int32),
                        pltpu.VMEM((_NCH, _NCH), jnp.float32),
                        pltpu.VMEM((128, 128), jnp.float32),
                        pltpu.VMEM((8, 8), jnp.float32)],
    )(y_i, hb_i, lb_i, memy_p)

    ridx = ridx2d.reshape(_B)
    x3 = x.reshape(_B, 1, _D)
    memx3 = mem_x.reshape(_CAP, 1, _D)
    out = pl.pallas_call(
        _scatter_body,
        grid_spec=pltpu.PrefetchScalarGridSpec(
            num_scalar_prefetch=1,
            grid=(_B,),
            in_specs=[pl.BlockSpec((1, 1, _D), lambda i, ridx_s: (i, 0, 0)),
                      pl.BlockSpec(memory_space=pl.ANY)],
            out_specs=pl.BlockSpec((1, 1, _D),
                                   lambda i, ridx_s: (ridx_s[i], 0, 0)),
        ),
        out_shape=jax.ShapeDtypeStruct((_CAP, 1, _D), x.dtype),
        input_output_aliases={2: 0},
    )(ridx, x3, memx3)

    mem_x_out = out.reshape(_CAP, _D)
    mem_y_out = memy_new.reshape(-1)[:_CAP].astype(ydt)
    return mem_x_out, mem_y_out
